# baseline (speedup 1.0000x reference)
import functools

import jax
import jax.numpy as jnp
from jax import lax
from jax.experimental import pallas as pl
from jax.experimental.pallas import tpu as pltpu
from jax.experimental.pallas import tpu_sc as plsc

VOCAB = 100000
EMBED = 64
BATCH = 1024

_NC = 2
_NS = 16
_NW = _NC * _NS
_B_PER_W = BATCH // _NW

_VBLK = 4096
_VGRID = (VOCAB + _VBLK - 1) // _VBLK


_NBUF = 12
_SLAB = 128


def _gather_rows(x, emb_t):
    mesh = plsc.VectorSubcoreMesh(core_axis_name="c", subcore_axis_name="s")

    @functools.partial(
        pl.kernel,
        mesh=mesh,
        out_type=jax.ShapeDtypeStruct((BATCH, EMBED), jnp.float32),
        scratch_types=[
            pltpu.VMEM((_B_PER_W,), jnp.int32),
            [pltpu.VMEM((EMBED, _SLAB), jnp.float32) for _ in range(_NBUF)],
            pltpu.VMEM((_B_PER_W, EMBED), jnp.float32),
            [pltpu.SemaphoreType.DMA for _ in range(_NBUF)],
        ],
        compiler_params=pltpu.CompilerParams(
            needs_layout_passes=False,
            skip_device_barrier=True,
            disable_bounds_checks=True,
        ),
    )
    def gather_kernel(idx_hbm, table_hbm, out_hbm, x_v, bufs, h_v, sems):
        wid = lax.axis_index("s") * _NC + lax.axis_index("c")
        base = wid * _B_PER_W
        pltpu.sync_copy(idx_hbm.at[pl.ds(base, _B_PER_W)], x_v)
        xvs = [x_v[pl.ds(t * 16, 16)] for t in range(_B_PER_W // 16)]

        def start(i):
            coff = lax.shift_right_logical(xvs[i // 16][i % 16], 7) * _SLAB
            return pltpu.async_copy(
                table_hbm.at[:, pl.ds(coff, _SLAB)], bufs[i % _NBUF], sems[i % _NBUF]
            )

        descs = [None] * _B_PER_W
        for i in range(min(_NBUF - 1, _B_PER_W)):
            descs[i] = start(i)
        for i in range(_B_PER_W):
            nxt = i + _NBUF - 1
            if nxt < _B_PER_W:
                descs[nxt] = start(nxt)
            descs[i].wait()
            lane = jnp.full((16,), xvs[i // 16][i % 16] & (_SLAB - 1), jnp.int32)
            for j in range(EMBED // 16):
                rows = lax.iota(jnp.int32, 16) + (j * 16)
                h_v[i, pl.ds(j * 16, 16)] = plsc.load_gather(
                    bufs[i % _NBUF], [rows, lane]
                )
        pltpu.sync_copy(h_v, out_hbm.at[pl.ds(base, _B_PER_W)])

    return gather_kernel(x, emb_t)


def _mm_body(w_ref, h_ref, b_ref, o_ref):
    acc = lax.dot_general(
        w_ref[...], h_ref[...],
        (((0,), (1,)), ((), ())),
        preferred_element_type=jnp.float32,
    )
    ones = jnp.ones((BATCH, 1), jnp.float32)
    bias = lax.dot_general(
        b_ref[...], ones,
        (((0,), (1,)), ((), ())),
        preferred_element_type=jnp.float32,
    )
    o_ref[...] = acc + bias


def _project(h, W_out, b_out):
    b2 = b_out.reshape(1, VOCAB)
    out_t = pl.pallas_call(
        _mm_body,
        grid=(_VGRID,),
        in_specs=[
            pl.BlockSpec((EMBED, _VBLK), lambda j: (0, j)),
            pl.BlockSpec((BATCH, EMBED), lambda j: (0, 0)),
            pl.BlockSpec((1, _VBLK), lambda j: (0, j)),
        ],
        out_specs=pl.BlockSpec((_VBLK, BATCH), lambda j: (j, 0)),
        out_shape=jax.ShapeDtypeStruct((VOCAB, BATCH), jnp.float32),
        compiler_params=pltpu.CompilerParams(vmem_limit_bytes=57_000_000),
    )(W_out, h, b2)
    return out_t.T


def kernel(x, emb_table, W_out, b_out):
    h = _gather_rows(x.astype(jnp.int32), emb_table.T)
    return _project(h, W_out, b_out)


# speedup vs baseline: 1.0069x; 1.0069x over previous
import functools

import jax
import jax.numpy as jnp
from jax import lax
from jax.experimental import pallas as pl
from jax.experimental.pallas import tpu as pltpu
from jax.experimental.pallas import tpu_sc as plsc

VOCAB = 100000
EMBED = 64
BATCH = 1024

_NC = 2
_NS = 16
_NW = _NC * _NS
_B_PER_W = BATCH // _NW

_VBLK = 4096
_VGRID = (VOCAB + _VBLK - 1) // _VBLK


_NBUF = 8
_SLAB = 128


def _gather_rows(x, emb_t):
    mesh = plsc.VectorSubcoreMesh(core_axis_name="c", subcore_axis_name="s")

    @functools.partial(
        pl.kernel,
        mesh=mesh,
        out_type=jax.ShapeDtypeStruct((BATCH, EMBED), jnp.float32),
        scratch_types=[
            pltpu.VMEM((_B_PER_W,), jnp.int32),
            [pltpu.VMEM((EMBED, _SLAB), jnp.float32) for _ in range(_NBUF)],
            pltpu.VMEM((_B_PER_W, EMBED), jnp.float32),
            [pltpu.SemaphoreType.DMA for _ in range(_NBUF)],
        ],
        compiler_params=pltpu.CompilerParams(
            needs_layout_passes=False,
            skip_device_barrier=True,
            disable_bounds_checks=True,
        ),
    )
    def gather_kernel(idx_hbm, table_hbm, out_hbm, x_v, bufs, h_v, sems):
        wid = lax.axis_index("s") * _NC + lax.axis_index("c")
        base = wid * _B_PER_W
        pltpu.sync_copy(idx_hbm.at[pl.ds(base, _B_PER_W)], x_v)
        xvs = [x_v[pl.ds(t * 16, 16)] for t in range(_B_PER_W // 16)]

        def start(i):
            coff = lax.shift_right_logical(xvs[i // 16][i % 16], 7) * _SLAB
            return pltpu.async_copy(
                table_hbm.at[:, pl.ds(coff, _SLAB)], bufs[i % _NBUF], sems[i % _NBUF]
            )

        descs = [None] * _B_PER_W
        for i in range(min(_NBUF - 1, _B_PER_W)):
            descs[i] = start(i)
        for i in range(_B_PER_W):
            nxt = i + _NBUF - 1
            if nxt < _B_PER_W:
                descs[nxt] = start(nxt)
            descs[i].wait()
            lane = jnp.full((16,), xvs[i // 16][i % 16] & (_SLAB - 1), jnp.int32)
            for j in range(EMBED // 16):
                rows = lax.iota(jnp.int32, 16) + (j * 16)
                h_v[i, pl.ds(j * 16, 16)] = plsc.load_gather(
                    bufs[i % _NBUF], [rows, lane]
                )
        pltpu.sync_copy(h_v, out_hbm.at[pl.ds(base, _B_PER_W)])

    return gather_kernel(x, emb_t)


def _mm_body(w_ref, h_ref, b_ref, o_ref):
    acc = lax.dot_general(
        w_ref[...], h_ref[...],
        (((0,), (1,)), ((), ())),
        preferred_element_type=jnp.float32,
    )
    ones = jnp.ones((BATCH, 1), jnp.float32)
    bias = lax.dot_general(
        b_ref[...], ones,
        (((0,), (1,)), ((), ())),
        preferred_element_type=jnp.float32,
    )
    o_ref[...] = acc + bias


def _project(h, W_out, b_out):
    b2 = b_out.reshape(1, VOCAB)
    out_t = pl.pallas_call(
        _mm_body,
        grid=(_VGRID,),
        in_specs=[
            pl.BlockSpec((EMBED, _VBLK), lambda j: (0, j)),
            pl.BlockSpec((BATCH, EMBED), lambda j: (0, 0)),
            pl.BlockSpec((1, _VBLK), lambda j: (0, j)),
        ],
        out_specs=pl.BlockSpec((_VBLK, BATCH), lambda j: (j, 0)),
        out_shape=jax.ShapeDtypeStruct((VOCAB, BATCH), jnp.float32),
        compiler_params=pltpu.CompilerParams(vmem_limit_bytes=57_000_000),
    )(W_out, h, b2)
    return out_t.T


def kernel(x, emb_table, W_out, b_out):
    h = _gather_rows(x.astype(jnp.int32), emb_table.T)
    return _project(h, W_out, b_out)
